# baseline (speedup 1.0000x reference)
import functools

import jax
import jax.numpy as jnp
from jax import lax
from jax.experimental import pallas as pl
from jax.experimental.pallas import tpu as pltpu
from jax.experimental.pallas import tpu_sc as plsc

NC, NS, L = 2, 16, 16
NW = NC * NS

A_ROWS, B_ROWS, D = 4096, 2048, 1024
A_PW = A_ROWS // NW
B_PW = B_ROWS // NW

R = 8
A_STEPS = A_PW // R
B_STEPS = B_PW // R
NSTEPS = A_STEPS + B_STEPS
NBUF = 14
UNROLL = 8


def _compute_chunk(buf):
    @pl.loop(0, R)
    def _(r):
        @plsc.parallel_loop(0, D, step=L, unroll=UNROLL)
        def _(c):
            sl = pl.ds(pl.multiple_of(c, L), L)
            buf[r, sl] = jnp.maximum(buf[r, sl] * 2.0 + 5.0, 0.0)


def _body(a_hbm, b_hbm, out_hbm, *scratch):
    bufs = scratch[:NBUF]
    isem = scratch[NBUF : 2 * NBUF]
    osem = scratch[2 * NBUF :]

    wid = lax.axis_index("s") * NC + lax.axis_index("c")
    a_base = pl.multiple_of(wid * A_PW, R)
    b_base = pl.multiple_of(wid * B_PW, R)

    def src_slice(t):
        if t < A_STEPS:
            return a_hbm.at[pl.ds(a_base + t * R, R)]
        tb = t - A_STEPS
        return b_hbm.at[pl.ds(b_base + tb * R, R)]

    def dst_slice(t):
        if t < A_STEPS:
            return out_hbm.at[pl.ds(a_base + t * R, R)]
        tb = t - A_STEPS
        return out_hbm.at[pl.ds(A_ROWS + b_base + tb * R, R)]

    for t in range(min(NBUF, NSTEPS)):
        pltpu.async_copy(src_slice(t), bufs[t % NBUF], isem[t % NBUF])

    for t in range(NSTEPS):
        s = t % NBUF
        pltpu.make_async_copy(src_slice(t), bufs[s], isem[s]).wait()
        _compute_chunk(bufs[s])
        pltpu.async_copy(bufs[s], dst_slice(t), osem[s])
        if t + NBUF < NSTEPS:
            pltpu.make_async_copy(bufs[s], dst_slice(t), osem[s]).wait()
            pltpu.async_copy(src_slice(t + NBUF), bufs[s], isem[s])

    for t in range(max(0, NSTEPS - NBUF), NSTEPS):
        s = t % NBUF
        pltpu.make_async_copy(bufs[s], dst_slice(t), osem[s]).wait()


def kernel(a, b):
    mesh = plsc.VectorSubcoreMesh(
        core_axis_name="c", subcore_axis_name="s", num_cores=NC, num_subcores=NS
    )
    out = pl.kernel(
        _body,
        out_type=jax.ShapeDtypeStruct((A_ROWS + B_ROWS, D), jnp.float32),
        mesh=mesh,
        compiler_params=pltpu.CompilerParams(
            disable_bounds_checks=True,
            disable_semaphore_checks=True,
            skip_device_barrier=True,
            use_tc_tiling_on_sc=True,
        ),
        scratch_types=(
            [pltpu.VMEM((R, D), jnp.float32)] * NBUF
            + [pltpu.SemaphoreType.DMA] * (2 * NBUF)
        ),
    )(a, b)
    return out


# speedup vs baseline: 1.0265x; 1.0265x over previous
import jax
import jax.numpy as jnp
from jax import lax
from jax.experimental import pallas as pl
from jax.experimental.pallas import tpu as pltpu
from jax.experimental.pallas import tpu_sc as plsc

NC, NS, L = 2, 16, 16
NW = NC * NS

A_ROWS, B_ROWS, D = 4096, 2048, 1024
A_PW = A_ROWS // NW
B_PW = B_ROWS // NW

R = 16
A_STEPS = A_PW // R
B_STEPS = B_PW // R
NSTEPS = A_STEPS + B_STEPS
NBUF = 7
UNROLL = 8


def _compute_chunk(buf):
    @pl.loop(0, R)
    def _(r):
        @plsc.parallel_loop(0, D, step=L, unroll=UNROLL)
        def _(c):
            sl = pl.ds(pl.multiple_of(c, L), L)
            buf[r, sl] = jnp.maximum(buf[r, sl] * 2.0 + 5.0, 0.0)


def _body(a_hbm, b_hbm, out_hbm, *scratch):
    bufs = scratch[:NBUF]
    isem = scratch[NBUF : 2 * NBUF]
    osem = scratch[2 * NBUF :]

    wid = lax.axis_index("s") * NC + lax.axis_index("c")
    a_base = pl.multiple_of(wid * A_PW, R)
    b_base = pl.multiple_of(wid * B_PW, R)

    def src_slice(t):
        if t < A_STEPS:
            return a_hbm.at[pl.ds(a_base + t * R, R)]
        tb = t - A_STEPS
        return b_hbm.at[pl.ds(b_base + tb * R, R)]

    def dst_slice(t):
        if t < A_STEPS:
            return out_hbm.at[pl.ds(a_base + t * R, R)]
        tb = t - A_STEPS
        return out_hbm.at[pl.ds(A_ROWS + b_base + tb * R, R)]

    for t in range(min(NBUF, NSTEPS)):
        pltpu.async_copy(src_slice(t), bufs[t % NBUF], isem[t % NBUF])

    for t in range(NSTEPS):
        s = t % NBUF
        pltpu.make_async_copy(src_slice(t), bufs[s], isem[s]).wait()
        _compute_chunk(bufs[s])
        pltpu.async_copy(bufs[s], dst_slice(t), osem[s])
        if t + NBUF < NSTEPS:
            pltpu.make_async_copy(bufs[s], dst_slice(t), osem[s]).wait()
            pltpu.async_copy(src_slice(t + NBUF), bufs[s], isem[s])

    for t in range(max(0, NSTEPS - NBUF), NSTEPS):
        s = t % NBUF
        pltpu.make_async_copy(bufs[s], dst_slice(t), osem[s]).wait()


def kernel(a, b):
    mesh = plsc.VectorSubcoreMesh(
        core_axis_name="c", subcore_axis_name="s", num_cores=NC, num_subcores=NS
    )
    out = pl.kernel(
        _body,
        out_type=jax.ShapeDtypeStruct((A_ROWS + B_ROWS, D), jnp.float32),
        mesh=mesh,
        compiler_params=pltpu.CompilerParams(
            disable_bounds_checks=True,
            disable_semaphore_checks=True,
            skip_device_barrier=True,
            use_tc_tiling_on_sc=True,
        ),
        scratch_types=(
            [pltpu.VMEM((R, D), jnp.float32)] * NBUF
            + [pltpu.SemaphoreType.DMA] * (2 * NBUF)
        ),
    )(a, b)
    return out
